# baseline (speedup 1.0000x reference)
import jax
import jax.numpy as jnp
from jax.experimental import pallas as pl
from jax.experimental.pallas import tpu as pltpu

_ALPHA = 1.702
_LIMIT = 7.0
_CHUNKS = 2


def _moe_body(ri_ref, rw_ref, x_ref, wgu_ref, bgu_ref, wd_ref, bd_ref,
              out_ref, sel_even_ref, sel_odd_ref):
    e = pl.program_id(0)
    j = pl.program_id(1)
    i2c = wgu_ref.shape[2]
    ic = i2c // 2

    @pl.when((e == 0) & (j == 0))
    def _init():
        out_ref[...] = jnp.zeros_like(out_ref)
        row = jax.lax.broadcasted_iota(jnp.int32, (i2c, ic), 0)
        col = jax.lax.broadcasted_iota(jnp.int32, (i2c, ic), 1)
        sel_even_ref[...] = (row == 2 * col).astype(jnp.float32)
        sel_odd_ref[...] = (row == 2 * col + 1).astype(jnp.float32)

    x = x_ref[...]
    gu = jnp.dot(x, wgu_ref[0], preferred_element_type=jnp.float32) + bgu_ref[0, 0]
    gate_full = jnp.minimum(gu, _LIMIT)
    up_full = jnp.clip(gu, -_LIMIT, _LIMIT)
    glu_full = gate_full * jax.nn.sigmoid(gate_full * _ALPHA)
    glu = jnp.dot(glu_full, sel_even_ref[...], preferred_element_type=jnp.float32)
    up = jnp.dot(up_full, sel_odd_ref[...], preferred_element_type=jnp.float32)
    gated = (up + 1.0) * glu
    out = jnp.dot(gated, wd_ref[0], preferred_element_type=jnp.float32)
    out = jnp.where(j == 0, out + bd_ref[0, 0], out)
    w = jnp.sum(rw_ref[...] * (ri_ref[...] == e).astype(jnp.float32), axis=1,
                keepdims=True)
    out_ref[...] += out * w


def kernel(hidden_states, router_indices, routing_weights, gate_up_proj,
           gate_up_proj_bias, down_proj, down_proj_bias):
    T, H = hidden_states.shape
    E, _, I2 = gate_up_proj.shape
    I = I2 // 2
    C = _CHUNKS
    I2c = I2 // C
    Ic = I // C

    bgu3 = gate_up_proj_bias.reshape(E, 1, I2)
    bd3 = down_proj_bias.reshape(E, 1, H)

    out = pl.pallas_call(
        _moe_body,
        grid=(E, C),
        in_specs=[
            pl.BlockSpec((T, router_indices.shape[1]), lambda e, j: (0, 0)),
            pl.BlockSpec((T, routing_weights.shape[1]), lambda e, j: (0, 0)),
            pl.BlockSpec((T, H), lambda e, j: (0, 0)),
            pl.BlockSpec((1, H, I2c), lambda e, j: (e, 0, j)),
            pl.BlockSpec((1, 1, I2c), lambda e, j: (e, 0, j)),
            pl.BlockSpec((1, Ic, H), lambda e, j: (e, j, 0)),
            pl.BlockSpec((1, 1, H), lambda e, j: (e, 0, 0)),
        ],
        out_specs=pl.BlockSpec((T, H), lambda e, j: (0, 0)),
        out_shape=jax.ShapeDtypeStruct((T, H), hidden_states.dtype),
        scratch_shapes=[
            pltpu.VMEM((I2c, Ic), jnp.float32),
            pltpu.VMEM((I2c, Ic), jnp.float32),
        ],
        compiler_params=pltpu.CompilerParams(
            dimension_semantics=("arbitrary", "arbitrary"),
        ),
    )(router_indices, routing_weights, hidden_states, gate_up_proj, bgu3,
      down_proj, bd3)
    return out


# speedup vs baseline: 1.0345x; 1.0345x over previous
import jax
import jax.numpy as jnp
from jax.experimental import pallas as pl
from jax.experimental.pallas import tpu as pltpu

_ALPHA = 1.702
_LIMIT = 7.0
_CHUNKS = 1


def _moe_body(ri_ref, rw_ref, x_ref, wgu_ref, bgu_ref, wd_ref, bd_ref,
              out_ref, sel_even_ref, sel_odd_ref):
    e = pl.program_id(0)
    j = pl.program_id(1)
    i2c = wgu_ref.shape[2]
    ic = i2c // 2

    @pl.when((e == 0) & (j == 0))
    def _init():
        out_ref[...] = jnp.zeros_like(out_ref)
        row = jax.lax.broadcasted_iota(jnp.int32, (i2c, ic), 0)
        col = jax.lax.broadcasted_iota(jnp.int32, (i2c, ic), 1)
        sel_even_ref[...] = (row == 2 * col).astype(jnp.float32)
        sel_odd_ref[...] = (row == 2 * col + 1).astype(jnp.float32)

    x = x_ref[...]
    gu = jnp.dot(x, wgu_ref[0], preferred_element_type=jnp.float32) + bgu_ref[0, 0]
    gate_full = jnp.minimum(gu, _LIMIT)
    up_full = jnp.clip(gu, -_LIMIT, _LIMIT)
    glu_full = gate_full * jax.nn.sigmoid(gate_full * _ALPHA)
    glu = jnp.dot(glu_full, sel_even_ref[...], preferred_element_type=jnp.float32)
    up = jnp.dot(up_full, sel_odd_ref[...], preferred_element_type=jnp.float32)
    gated = (up + 1.0) * glu
    out = jnp.dot(gated, wd_ref[0], preferred_element_type=jnp.float32)
    out = jnp.where(j == 0, out + bd_ref[0, 0], out)
    w = jnp.sum(rw_ref[...] * (ri_ref[...] == e).astype(jnp.float32), axis=1,
                keepdims=True)
    out_ref[...] += out * w


def kernel(hidden_states, router_indices, routing_weights, gate_up_proj,
           gate_up_proj_bias, down_proj, down_proj_bias):
    T, H = hidden_states.shape
    E, _, I2 = gate_up_proj.shape
    I = I2 // 2
    C = _CHUNKS
    I2c = I2 // C
    Ic = I // C

    bgu3 = gate_up_proj_bias.reshape(E, 1, I2)
    bd3 = down_proj_bias.reshape(E, 1, H)

    out = pl.pallas_call(
        _moe_body,
        grid=(E, C),
        in_specs=[
            pl.BlockSpec((T, router_indices.shape[1]), lambda e, j: (0, 0)),
            pl.BlockSpec((T, routing_weights.shape[1]), lambda e, j: (0, 0)),
            pl.BlockSpec((T, H), lambda e, j: (0, 0)),
            pl.BlockSpec((1, H, I2c), lambda e, j: (e, 0, j)),
            pl.BlockSpec((1, 1, I2c), lambda e, j: (e, 0, j)),
            pl.BlockSpec((1, Ic, H), lambda e, j: (e, j, 0)),
            pl.BlockSpec((1, 1, H), lambda e, j: (e, 0, 0)),
        ],
        out_specs=pl.BlockSpec((T, H), lambda e, j: (0, 0)),
        out_shape=jax.ShapeDtypeStruct((T, H), hidden_states.dtype),
        scratch_shapes=[
            pltpu.VMEM((I2c, Ic), jnp.float32),
            pltpu.VMEM((I2c, Ic), jnp.float32),
        ],
        compiler_params=pltpu.CompilerParams(
            dimension_semantics=("arbitrary", "arbitrary"),
        ),
    )(router_indices, routing_weights, hidden_states, gate_up_proj, bgu3,
      down_proj, bd3)
    return out


# speedup vs baseline: 1.0459x; 1.0111x over previous
import jax
import jax.numpy as jnp
from jax.experimental import pallas as pl
from jax.experimental.pallas import tpu as pltpu

_ALPHA = 1.702
_LIMIT = 7.0


def _moe_body(ri_ref, rw_ref, x_ref, wgu0_ref, wgu1_ref, bgu_ref,
              wd0_ref, wd1_ref, bd_ref, out_ref, sel_even_ref, sel_odd_ref):
    e = pl.program_id(0)
    i2 = wgu0_ref.shape[3]
    i = i2 // 2
    hh = wgu0_ref.shape[2]

    @pl.when(e == 0)
    def _init():
        out_ref[...] = jnp.zeros_like(out_ref)
        row = jax.lax.broadcasted_iota(jnp.int32, (i2, i), 0)
        col = jax.lax.broadcasted_iota(jnp.int32, (i2, i), 1)
        sel_even_ref[...] = (row == 2 * col).astype(jnp.float32)
        sel_odd_ref[...] = (row == 2 * col + 1).astype(jnp.float32)

    x = x_ref[...]
    gu = (jnp.dot(x[:, :hh], wgu0_ref[0, 0], preferred_element_type=jnp.float32)
          + jnp.dot(x[:, hh:], wgu1_ref[0, 0], preferred_element_type=jnp.float32)
          + bgu_ref[0, 0])
    gate_full = jnp.minimum(gu, _LIMIT)
    up_full = jnp.clip(gu, -_LIMIT, _LIMIT)
    glu_full = gate_full * jax.nn.sigmoid(gate_full * _ALPHA)
    glu = jnp.dot(glu_full, sel_even_ref[...], preferred_element_type=jnp.float32)
    up = jnp.dot(up_full, sel_odd_ref[...], preferred_element_type=jnp.float32)
    gated = (up + 1.0) * glu
    out = (jnp.dot(gated[:, :i // 2], wd0_ref[0, 0], preferred_element_type=jnp.float32)
           + jnp.dot(gated[:, i // 2:], wd1_ref[0, 0], preferred_element_type=jnp.float32)
           + bd_ref[0, 0])
    w = jnp.sum(rw_ref[...] * (ri_ref[...] == e).astype(jnp.float32), axis=1,
                keepdims=True)
    out_ref[...] += out * w


def kernel(hidden_states, router_indices, routing_weights, gate_up_proj,
           gate_up_proj_bias, down_proj, down_proj_bias):
    T, H = hidden_states.shape
    E, _, I2 = gate_up_proj.shape
    I = I2 // 2

    wgu4 = gate_up_proj.reshape(E, 2, H // 2, I2)
    wd4 = down_proj.reshape(E, 2, I // 2, H)
    bgu3 = gate_up_proj_bias.reshape(E, 1, I2)
    bd3 = down_proj_bias.reshape(E, 1, H)

    out = pl.pallas_call(
        _moe_body,
        grid=(E,),
        in_specs=[
            pl.BlockSpec((T, router_indices.shape[1]), lambda e: (0, 0)),
            pl.BlockSpec((T, routing_weights.shape[1]), lambda e: (0, 0)),
            pl.BlockSpec((T, H), lambda e: (0, 0)),
            pl.BlockSpec((1, 1, H // 2, I2), lambda e: (e, 0, 0, 0)),
            pl.BlockSpec((1, 1, H // 2, I2), lambda e: (e, 1, 0, 0)),
            pl.BlockSpec((1, 1, I2), lambda e: (e, 0, 0)),
            pl.BlockSpec((1, 1, I // 2, H), lambda e: (e, 0, 0, 0)),
            pl.BlockSpec((1, 1, I // 2, H), lambda e: (e, 1, 0, 0)),
            pl.BlockSpec((1, 1, H), lambda e: (e, 0, 0)),
        ],
        out_specs=pl.BlockSpec((T, H), lambda e: (0, 0)),
        out_shape=jax.ShapeDtypeStruct((T, H), hidden_states.dtype),
        scratch_shapes=[
            pltpu.VMEM((I2, I), jnp.float32),
            pltpu.VMEM((I2, I), jnp.float32),
        ],
        compiler_params=pltpu.CompilerParams(
            dimension_semantics=("arbitrary",),
        ),
    )(router_indices, routing_weights, hidden_states, wgu4, wgu4, bgu3,
      wd4, wd4, bd3)
    return out


# speedup vs baseline: 1.0593x; 1.0128x over previous
import jax
import jax.numpy as jnp
from jax.experimental import pallas as pl
from jax.experimental.pallas import tpu as pltpu

_ALPHA = 1.702
_LIMIT = 7.0
_GU_SPLIT = 4
_D_SPLIT = 4


def _moe_body(ri_ref, rw_ref, x_ref, *refs):
    wgu_refs = refs[:_GU_SPLIT]
    bgu_ref = refs[_GU_SPLIT]
    wd_refs = refs[_GU_SPLIT + 1:_GU_SPLIT + 1 + _D_SPLIT]
    bd_ref = refs[_GU_SPLIT + 1 + _D_SPLIT]
    out_ref = refs[_GU_SPLIT + _D_SPLIT + 2]
    sel_even_ref = refs[_GU_SPLIT + _D_SPLIT + 3]
    sel_odd_ref = refs[_GU_SPLIT + _D_SPLIT + 4]

    e = pl.program_id(0)
    i2 = wgu_refs[0].shape[3]
    i = i2 // 2
    hh = wgu_refs[0].shape[2]
    ih = wd_refs[0].shape[2]

    @pl.when(e == 0)
    def _init():
        out_ref[...] = jnp.zeros_like(out_ref)
        row = jax.lax.broadcasted_iota(jnp.int32, (i2, i), 0)
        col = jax.lax.broadcasted_iota(jnp.int32, (i2, i), 1)
        sel_even_ref[...] = (row == 2 * col).astype(jnp.float32)
        sel_odd_ref[...] = (row == 2 * col + 1).astype(jnp.float32)

    x = x_ref[...]
    gu = bgu_ref[0, 0]
    for q, wref in enumerate(wgu_refs):
        gu = gu + jnp.dot(x[:, q * hh:(q + 1) * hh], wref[0, 0],
                          preferred_element_type=jnp.float32)
    gate_full = jnp.minimum(gu, _LIMIT)
    up_full = jnp.clip(gu, -_LIMIT, _LIMIT)
    glu_full = gate_full * jax.nn.sigmoid(gate_full * _ALPHA)
    glu = jnp.dot(glu_full, sel_even_ref[...], preferred_element_type=jnp.float32)
    up = jnp.dot(up_full, sel_odd_ref[...], preferred_element_type=jnp.float32)
    gated = (up + 1.0) * glu
    out = bd_ref[0, 0]
    for q, wref in enumerate(wd_refs):
        out = out + jnp.dot(gated[:, q * ih:(q + 1) * ih], wref[0, 0],
                            preferred_element_type=jnp.float32)
    w = jnp.sum(rw_ref[...] * (ri_ref[...] == e).astype(jnp.float32), axis=1,
                keepdims=True)
    out_ref[...] += out * w


def kernel(hidden_states, router_indices, routing_weights, gate_up_proj,
           gate_up_proj_bias, down_proj, down_proj_bias):
    T, H = hidden_states.shape
    E, _, I2 = gate_up_proj.shape
    I = I2 // 2
    G, D = _GU_SPLIT, _D_SPLIT

    wgu4 = gate_up_proj.reshape(E, G, H // G, I2)
    wd4 = down_proj.reshape(E, D, I // D, H)
    bgu3 = gate_up_proj_bias.reshape(E, 1, I2)
    bd3 = down_proj_bias.reshape(E, 1, H)

    def gu_spec(q):
        return pl.BlockSpec((1, 1, H // G, I2), lambda e, q=q: (e, q, 0, 0))

    def d_spec(q):
        return pl.BlockSpec((1, 1, I // D, H), lambda e, q=q: (e, q, 0, 0))

    out = pl.pallas_call(
        _moe_body,
        grid=(E,),
        in_specs=[
            pl.BlockSpec((T, router_indices.shape[1]), lambda e: (0, 0)),
            pl.BlockSpec((T, routing_weights.shape[1]), lambda e: (0, 0)),
            pl.BlockSpec((T, H), lambda e: (0, 0)),
        ] + [gu_spec(q) for q in range(G)]
        + [pl.BlockSpec((1, 1, I2), lambda e: (e, 0, 0))]
        + [d_spec(q) for q in range(D)]
        + [pl.BlockSpec((1, 1, H), lambda e: (e, 0, 0))],
        out_specs=pl.BlockSpec((T, H), lambda e: (0, 0)),
        out_shape=jax.ShapeDtypeStruct((T, H), hidden_states.dtype),
        scratch_shapes=[
            pltpu.VMEM((I2, I), jnp.float32),
            pltpu.VMEM((I2, I), jnp.float32),
        ],
        compiler_params=pltpu.CompilerParams(
            dimension_semantics=("arbitrary",),
        ),
    )(router_indices, routing_weights, hidden_states,
      *([wgu4] * G), bgu3, *([wd4] * D), bd3)
    return out


# speedup vs baseline: 1.1002x; 1.0386x over previous
import jax
import jax.numpy as jnp
from jax.experimental import pallas as pl
from jax.experimental.pallas import tpu as pltpu

_ALPHA = 1.702
_LIMIT = 7.0
_GU_SPLIT = 8
_D_SPLIT = 8


def _moe_body(ri_ref, rw_ref, x_ref, *refs):
    wgu_refs = refs[:_GU_SPLIT]
    bgu_ref = refs[_GU_SPLIT]
    wd_refs = refs[_GU_SPLIT + 1:_GU_SPLIT + 1 + _D_SPLIT]
    bd_ref = refs[_GU_SPLIT + 1 + _D_SPLIT]
    out_ref = refs[_GU_SPLIT + _D_SPLIT + 2]
    sel_even_ref = refs[_GU_SPLIT + _D_SPLIT + 3]
    sel_odd_ref = refs[_GU_SPLIT + _D_SPLIT + 4]

    e = pl.program_id(0)
    i2 = wgu_refs[0].shape[3]
    i = i2 // 2
    hh = wgu_refs[0].shape[2]
    ih = wd_refs[0].shape[2]

    @pl.when(e == 0)
    def _init():
        out_ref[...] = jnp.zeros_like(out_ref)
        row = jax.lax.broadcasted_iota(jnp.int32, (i2, i), 0)
        col = jax.lax.broadcasted_iota(jnp.int32, (i2, i), 1)
        sel_even_ref[...] = (row == 2 * col).astype(jnp.float32)
        sel_odd_ref[...] = (row == 2 * col + 1).astype(jnp.float32)

    x = x_ref[...]
    gu = bgu_ref[0, 0]
    for q, wref in enumerate(wgu_refs):
        gu = gu + jnp.dot(x[:, q * hh:(q + 1) * hh], wref[0, 0],
                          preferred_element_type=jnp.float32)
    gate_full = jnp.minimum(gu, _LIMIT)
    up_full = jnp.clip(gu, -_LIMIT, _LIMIT)
    glu_full = gate_full * jax.nn.sigmoid(gate_full * _ALPHA)
    glu = jnp.dot(glu_full, sel_even_ref[...], preferred_element_type=jnp.float32)
    up = jnp.dot(up_full, sel_odd_ref[...], preferred_element_type=jnp.float32)
    gated = (up + 1.0) * glu
    out = bd_ref[0, 0]
    for q, wref in enumerate(wd_refs):
        out = out + jnp.dot(gated[:, q * ih:(q + 1) * ih], wref[0, 0],
                            preferred_element_type=jnp.float32)
    w = jnp.sum(rw_ref[...] * (ri_ref[...] == e).astype(jnp.float32), axis=1,
                keepdims=True)
    out_ref[...] += out * w


def kernel(hidden_states, router_indices, routing_weights, gate_up_proj,
           gate_up_proj_bias, down_proj, down_proj_bias):
    T, H = hidden_states.shape
    E, _, I2 = gate_up_proj.shape
    I = I2 // 2
    G, D = _GU_SPLIT, _D_SPLIT

    wgu4 = gate_up_proj.reshape(E, G, H // G, I2)
    wd4 = down_proj.reshape(E, D, I // D, H)
    bgu3 = gate_up_proj_bias.reshape(E, 1, I2)
    bd3 = down_proj_bias.reshape(E, 1, H)

    def gu_spec(q):
        return pl.BlockSpec((1, 1, H // G, I2), lambda e, q=q: (e, q, 0, 0))

    def d_spec(q):
        return pl.BlockSpec((1, 1, I // D, H), lambda e, q=q: (e, q, 0, 0))

    out = pl.pallas_call(
        _moe_body,
        grid=(E,),
        in_specs=[
            pl.BlockSpec((T, router_indices.shape[1]), lambda e: (0, 0)),
            pl.BlockSpec((T, routing_weights.shape[1]), lambda e: (0, 0)),
            pl.BlockSpec((T, H), lambda e: (0, 0)),
        ] + [gu_spec(q) for q in range(G)]
        + [pl.BlockSpec((1, 1, I2), lambda e: (e, 0, 0))]
        + [d_spec(q) for q in range(D)]
        + [pl.BlockSpec((1, 1, H), lambda e: (e, 0, 0))],
        out_specs=pl.BlockSpec((T, H), lambda e: (0, 0)),
        out_shape=jax.ShapeDtypeStruct((T, H), hidden_states.dtype),
        scratch_shapes=[
            pltpu.VMEM((I2, I), jnp.float32),
            pltpu.VMEM((I2, I), jnp.float32),
        ],
        compiler_params=pltpu.CompilerParams(
            dimension_semantics=("arbitrary",),
        ),
    )(router_indices, routing_weights, hidden_states,
      *([wgu4] * G), bgu3, *([wd4] * D), bd3)
    return out
